# baseline (speedup 1.0000x reference)
import jax
import jax.numpy as jnp
from jax.experimental import pallas as pl
from jax.experimental.pallas import tpu as pltpu


def _gap_mxu_kernel(x_ref, s_ref, o_ref):
    o_ref[...] = jax.lax.dot_general(
        x_ref[...],
        s_ref[...],
        dimension_numbers=(((1,), (0,)), ((), ())),
        preferred_element_type=jnp.float32,
    ).astype(o_ref.dtype)


@jax.jit
def _global_avg_pool_2d(x):
    N, C, H, W = x.shape
    R, HW = N * C, H * W
    inv_hw = 1.0 / float(HW)

    if R % 128 == 0:
        G = 128
    elif R % 8 == 0:
        G = 8
    else:
        G = 1
    M, K = R // G, G * HW

    x2 = x.reshape(M, K)

    s = jnp.where(
        (jax.lax.broadcasted_iota(jnp.int32, (K, G), 0) // HW)
        == jax.lax.broadcasted_iota(jnp.int32, (K, G), 1),
        jnp.float32(inv_hw),
        jnp.float32(0.0),
    )

    Mt = min(M, 256)
    out = pl.pallas_call(
        _gap_mxu_kernel,
        out_shape=jax.ShapeDtypeStruct((M, G), x.dtype),
        grid=(pl.cdiv(M, Mt),),
        in_specs=[
            pl.BlockSpec((Mt, K), lambda m: (m, 0)),
            pl.BlockSpec((K, G), lambda m: (0, 0)),
        ],
        out_specs=pl.BlockSpec((Mt, G), lambda m: (m, 0)),
        compiler_params=pltpu.CompilerParams(
            dimension_semantics=("parallel",),
            vmem_limit_bytes=48 * 1024 * 1024,
        ),
        cost_estimate=pl.CostEstimate(
            flops=2 * M * K * G,
            transcendentals=0,
            bytes_accessed=(M * K + K * G + M * G) * x.dtype.itemsize,
        ),
    )(x2, s)

    return out.reshape(N, C, 1, 1)


def kernel(x):
    return _global_avg_pool_2d(x)


# speedup vs baseline: 33.7737x; 33.7737x over previous
import functools

import jax
import jax.numpy as jnp
from jax.experimental import pallas as pl
from jax.experimental.pallas import tpu as pltpu


def _gap_hw_major_kernel(x_ref, o_ref, *, inv_hw):
    x = x_ref[...].astype(jnp.float32)
    o_ref[...] = (jnp.sum(x, axis=(0, 1)) * inv_hw).astype(o_ref.dtype)


@jax.jit
def _global_avg_pool_2d(x):
    N, C, H, W = x.shape
    inv_hw = 1.0 / float(H * W)

    xt = x.transpose(2, 3, 0, 1)

    nt = N
    itemsize = x.dtype.itemsize
    while nt > 8 and H * W * nt * C * itemsize > 8 * 1024 * 1024:
        nt //= 2

    out = pl.pallas_call(
        functools.partial(_gap_hw_major_kernel, inv_hw=inv_hw),
        out_shape=jax.ShapeDtypeStruct((N, C), x.dtype),
        grid=(pl.cdiv(N, nt),),
        in_specs=[pl.BlockSpec((H, W, nt, C), lambda i: (0, 0, i, 0))],
        out_specs=pl.BlockSpec((nt, C), lambda i: (i, 0)),
        compiler_params=pltpu.CompilerParams(
            dimension_semantics=("parallel",),
            vmem_limit_bytes=48 * 1024 * 1024,
        ),
        cost_estimate=pl.CostEstimate(
            flops=N * C * H * W,
            transcendentals=0,
            bytes_accessed=(N * C * H * W + N * C) * itemsize,
        ),
    )(xt)

    return out.reshape(N, C, 1, 1)


def kernel(x):
    return _global_avg_pool_2d(x)
